# speedup vs baseline: 207.1748x; 207.1748x over previous
import functools

import jax
import jax.numpy as jnp
from jax.experimental import pallas as pl
from jax.experimental.pallas import tpu as pltpu

B = 4
C3 = 30
C2 = 10
H = 512
W = 512
TH = 64


def _body(map_ref, w_ref, l2_ref, l3_ref, out_ref):
    b = pl.program_id(0)
    h = pl.program_id(1)
    nh = pl.num_programs(1)

    @pl.when((b == 0) & (h == 0))
    def _init():
        out_ref[0, 0] = 0.0

    l3max = l3_ref[0, 0]
    mapped = jnp.full((TH, W), map_ref[0], dtype=jnp.int32)
    for k in range(1, C3):
        v = l3_ref[0, k]
        upd = v > l3max
        l3max = jnp.where(upd, v, l3max)
        mapped = jnp.where(upd, map_ref[k], mapped)

    l2max = l2_ref[0, 0]
    idx2 = jnp.zeros((TH, W), dtype=jnp.int32)
    for k in range(1, C2):
        v = l2_ref[0, k]
        upd = v > l2max
        l2max = jnp.where(upd, v, l2max)
        idx2 = jnp.where(upd, k, idx2)

    out_ref[0, 0] += jnp.sum((mapped != idx2).astype(jnp.float32))

    @pl.when((b == pl.num_programs(0) - 1) & (h == nh - 1))
    def _finish():
        out_ref[0, 0] = out_ref[0, 0] * w_ref[0] * (1.0 / (B * H * W))


@jax.jit
def _loss(level2_pred, level3_pred, mapping, weight):
    out = pl.pallas_call(
        _body,
        grid=(B, H // TH),
        in_specs=[
            pl.BlockSpec(memory_space=pltpu.SMEM),
            pl.BlockSpec(memory_space=pltpu.SMEM),
            pl.BlockSpec((1, C2, TH, W), lambda b, h: (b, 0, h, 0)),
            pl.BlockSpec((1, C3, TH, W), lambda b, h: (b, 0, h, 0)),
        ],
        out_specs=pl.BlockSpec(memory_space=pltpu.SMEM),
        out_shape=jax.ShapeDtypeStruct((1, 1), jnp.float32),
    )(mapping, weight.reshape(1), level2_pred, level3_pred)
    return out.reshape(())


def kernel(level2_pred, level3_pred, mapping, weight):
    return _loss(level2_pred, level3_pred, mapping,
                 jnp.asarray(weight, jnp.float32))
